# baseline (speedup 1.0000x reference)
import functools
import math

import jax
import jax.numpy as jnp
from jax import lax
from jax.experimental import pallas as pl
from jax.experimental.pallas import tpu as pltpu
from jax.experimental.pallas import tpu_sc as plsc

NC, NS, L = 2, 16, 16
NW = NC * NS


def _make_sc_aggregate(r_pad, d, nch, chunk):
    rows_per_tile = r_pad // NS

    mesh = plsc.VectorSubcoreMesh(core_axis_name="c", subcore_axis_name="s")

    @functools.partial(
        pl.kernel,
        out_type=jax.ShapeDtypeStruct((NC, r_pad, d), jnp.float32),
        mesh=mesh,
        scratch_types=[
            pltpu.VMEM((4, chunk), jnp.int32),
            pltpu.VMEM((4, chunk), jnp.int32),
            pltpu.VMEM((4, chunk), jnp.int32),
            pltpu.VMEM((4, chunk), jnp.float32),
            pltpu.VMEM((chunk, d), jnp.float32),
            pltpu.VMEM((chunk, d), jnp.float32),
            pltpu.VMEM((chunk, d), jnp.float32),
            pltpu.VMEM((chunk, d), jnp.float32),
            pltpu.VMEM_SHARED((r_pad, d), jnp.float32),
            pltpu.SemaphoreType.DMA,
            pltpu.SemaphoreType.DMA,
            pltpu.SemaphoreType.DMA,
            pltpu.SemaphoreType.DMA,
            pltpu.SemaphoreType.DMA,
            pltpu.SemaphoreType.DMA,
            pltpu.SemaphoreType.DMA,
            pltpu.SemaphoreType.DMA,
            pltpu.SemaphoreType.DMA,
            pltpu.SemaphoreType.DMA,
            pltpu.SemaphoreType.DMA,
            pltpu.SemaphoreType.DMA,
        ],
    )
    def sc_kernel(x2_hbm, src_hbm, dst_hbm, val_hbm, zeros_hbm, part_hbm,
                  sring, dring, ldst, vring, rows0, rows1, rows2, rows3,
                  acc, ssem0, ssem1, ssem2, ssem3, gsem0, gsem1, gsem2,
                  gsem3, csem0, csem1, csem2, csem3):
        c = lax.axis_index("c")
        s = lax.axis_index("s")
        wid = s * NC + c
        bufs = (rows0, rows1, rows2, rows3)
        ssems = (ssem0, ssem1, ssem2, ssem3)
        gsems = (gsem0, gsem1, gsem2, gsem3)
        csems = (csem0, csem1, csem2, csem3)

        r0 = pl.multiple_of(s * rows_per_tile, rows_per_tile)
        pltpu.sync_copy(zeros_hbm.at[pl.ds(r0, rows_per_tile)],
                        acc.at[pl.ds(r0, rows_per_tile)])
        plsc.subcore_barrier()

        def start_slab(j, p):
            pltpu.async_copy(src_hbm.at[wid, j], sring.at[p], ssems[p])
            pltpu.async_copy(dst_hbm.at[wid, j], dring.at[p], ssems[p])
            pltpu.async_copy(val_hbm.at[wid, j], vring.at[p], ssems[p])

        def wait_slab(j, p):
            pltpu.make_async_copy(src_hbm.at[wid, j], sring.at[p],
                                  ssems[p]).wait()
            pltpu.make_async_copy(dst_hbm.at[wid, j], dring.at[p],
                                  ssems[p]).wait()
            pltpu.make_async_copy(val_hbm.at[wid, j], vring.at[p],
                                  ssems[p]).wait()

        def start_gather(p):
            pltpu.async_copy(x2_hbm.at[sring.at[p]], bufs[p], gsems[p])

        def wait_gather(p):
            pltpu.make_async_copy(x2_hbm.at[sring.at[p]], bufs[p],
                                  gsems[p]).wait()

        def start_scatter(p):
            pltpu.async_copy(bufs[p], acc.at[ldst.at[p]], csems[p],
                             add=True)

        def wait_scatter(p):
            pltpu.make_async_copy(bufs[p], acc.at[ldst.at[p]],
                                  csems[p]).wait()

        def scale(p):
            buf = bufs[p]
            for gq in range(chunk // L):
                vals16 = vring[p, pl.ds(gq * L, L)]
                for i in range(L):
                    v = vals16[i]
                    row = gq * L + i
                    for g in range(d // L):
                        sl = pl.ds(g * L, L)
                        buf[row, sl] = buf[row, sl] * v

        def quad_body(jj, carry):
            for p in range(4):
                j = jj * 4 + p
                wait_gather(p)
                for q in range(chunk // L):
                    sl = pl.ds(q * L, L)
                    ldst[p, sl] = dring[p, sl]
                p2 = (p + 2) % 4

                @pl.when(j + 2 < nch)
                def _():
                    wait_slab(j + 2, p2)

                    @pl.when(j >= 2)
                    def _():
                        wait_scatter(p2)
                    start_gather(p2)
                scale(p)
                start_scatter(p)
                @pl.when(j + 4 < nch)
                def _():
                    start_slab(j + 4, p)
            return carry

        for p in range(4):
            start_slab(p, p)
        wait_slab(0, 0)
        start_gather(0)
        wait_slab(1, 1)
        start_gather(1)
        lax.fori_loop(0, nch // 4, quad_body, 0)
        for p in range(4):
            wait_scatter(p)

        plsc.subcore_barrier()
        pltpu.sync_copy(acc.at[pl.ds(r0, rows_per_tile)],
                        part_hbm.at[c, pl.ds(r0, rows_per_tile)])

    return sc_kernel


def _tc_body(p_ref, w_ref, b_ref, g_ref, be_ref, o_ref):
    z = lax.dot_general(p_ref[0] + p_ref[1], w_ref[...],
                        (((1,), (1,)), ((), ())),
                        preferred_element_type=jnp.float32)
    z = z + b_ref[...]
    mu = jnp.mean(z, axis=-1, keepdims=True)
    zc = z - mu
    var = jnp.mean(zc * zc, axis=-1, keepdims=True)
    zn = zc * lax.rsqrt(var + 1e-5) * g_ref[...] + be_ref[...]
    o_ref[...] = zn * 0.5 * (1.0 + lax.erf(zn * (1.0 / math.sqrt(2.0))))


def kernel(x, edge_index, edge_values, W, b, gamma, beta):
    B, n, d_in = x.shape
    d = B * d_in
    d_out = W.shape[0]
    e = edge_values.shape[0]
    x2 = jnp.transpose(x.astype(jnp.float32), (1, 0, 2)).reshape(n, d)

    chunk = 64
    nch = -(-(-(-e // (NW * chunk))) // 4) * 4
    e_pad = NW * nch * chunk
    src = jnp.concatenate(
        [edge_index[1], jnp.zeros((e_pad - e,), jnp.int32)]).reshape(
            NW, nch, chunk)
    dst = jnp.concatenate(
        [edge_index[0], jnp.zeros((e_pad - e,), jnp.int32)]).reshape(
            NW, nch, chunk)
    val = jnp.concatenate(
        [edge_values, jnp.zeros((e_pad - e,), jnp.float32)]).reshape(
            NW, nch, chunk)

    r_pad = ((n + NS * 8 - 1) // (NS * 8)) * (NS * 8)
    zeros = jnp.zeros((r_pad, d), jnp.float32)

    sc = _make_sc_aggregate(r_pad, d, nch, chunk)
    partials = sc(x2, src, dst, val, zeros)

    blk = 2000
    out = pl.pallas_call(
        _tc_body,
        grid=(n // blk,),
        in_specs=[
            pl.BlockSpec((NC, blk, d), lambda i: (0, i, 0)),
            pl.BlockSpec((d_out, d), lambda i: (0, 0)),
            pl.BlockSpec((1, d_out), lambda i: (0, 0)),
            pl.BlockSpec((1, d_out), lambda i: (0, 0)),
            pl.BlockSpec((1, d_out), lambda i: (0, 0)),
        ],
        out_specs=pl.BlockSpec((blk, d_out), lambda i: (i, 0)),
        out_shape=jax.ShapeDtypeStruct((n, d_out), jnp.float32),
    )(partials, W, b.reshape(1, d_out), gamma.reshape(1, d_out),
      beta.reshape(1, d_out))

    return out.reshape(n, B, d_out).transpose(1, 0, 2)


# speedup vs baseline: 1.4868x; 1.4868x over previous
import functools
import math

import jax
import jax.numpy as jnp
from jax import lax
from jax.experimental import pallas as pl
from jax.experimental.pallas import tpu as pltpu
from jax.experimental.pallas import tpu_sc as plsc

NC, NS, L = 2, 16, 16
NW = NC * NS


def _make_sc_aggregate(r_pad, d, nch, chunk):
    rows_per_tile = r_pad // NS

    mesh = plsc.VectorSubcoreMesh(core_axis_name="c", subcore_axis_name="s")

    @functools.partial(
        pl.kernel,
        out_type=jax.ShapeDtypeStruct((NC, r_pad, d), jnp.float32),
        mesh=mesh,
        scratch_types=[
            pltpu.VMEM((4, chunk), jnp.int32),
            pltpu.VMEM((4, chunk), jnp.int32),
            pltpu.VMEM((4, chunk), jnp.int32),
            pltpu.VMEM((4, chunk), jnp.float32),
            pltpu.VMEM((chunk, d), jnp.float32),
            pltpu.VMEM((chunk, d), jnp.float32),
            pltpu.VMEM((chunk, d), jnp.float32),
            pltpu.VMEM((chunk, d), jnp.float32),
            pltpu.VMEM_SHARED((r_pad, d), jnp.float32),
            pltpu.SemaphoreType.DMA,
            pltpu.SemaphoreType.DMA,
            pltpu.SemaphoreType.DMA,
            pltpu.SemaphoreType.DMA,
            pltpu.SemaphoreType.DMA,
            pltpu.SemaphoreType.DMA,
            pltpu.SemaphoreType.DMA,
            pltpu.SemaphoreType.DMA,
            pltpu.SemaphoreType.DMA,
            pltpu.SemaphoreType.DMA,
            pltpu.SemaphoreType.DMA,
            pltpu.SemaphoreType.DMA,
        ],
    )
    def sc_kernel(x2_hbm, src_hbm, dst_hbm, val_hbm, zeros_hbm, part_hbm,
                  sring, dring, ldst, vring, rows0, rows1, rows2, rows3,
                  acc, ssem0, ssem1, ssem2, ssem3, gsem0, gsem1, gsem2,
                  gsem3, csem0, csem1, csem2, csem3):
        c = lax.axis_index("c")
        s = lax.axis_index("s")
        wid = s * NC + c
        bufs = (rows0, rows1, rows2, rows3)
        ssems = (ssem0, ssem1, ssem2, ssem3)
        gsems = (gsem0, gsem1, gsem2, gsem3)
        csems = (csem0, csem1, csem2, csem3)

        r0 = pl.multiple_of(s * rows_per_tile, rows_per_tile)
        pltpu.sync_copy(zeros_hbm.at[pl.ds(r0, rows_per_tile)],
                        acc.at[pl.ds(r0, rows_per_tile)])
        plsc.subcore_barrier()

        def start_slab(j, p):
            pltpu.async_copy(src_hbm.at[wid, j], sring.at[p], ssems[p])
            pltpu.async_copy(dst_hbm.at[wid, j], dring.at[p], ssems[p])
            pltpu.async_copy(val_hbm.at[wid, j], vring.at[p], ssems[p])

        def wait_slab(j, p):
            pltpu.make_async_copy(src_hbm.at[wid, j], sring.at[p],
                                  ssems[p]).wait()
            pltpu.make_async_copy(dst_hbm.at[wid, j], dring.at[p],
                                  ssems[p]).wait()
            pltpu.make_async_copy(val_hbm.at[wid, j], vring.at[p],
                                  ssems[p]).wait()

        def start_gather(p):
            pltpu.async_copy(x2_hbm.at[sring.at[p]], bufs[p], gsems[p])

        def wait_gather(p):
            pltpu.make_async_copy(x2_hbm.at[sring.at[p]], bufs[p],
                                  gsems[p]).wait()

        def start_scatter(p):
            pltpu.async_copy(bufs[p], acc.at[ldst.at[p]], csems[p],
                             add=True)

        def wait_scatter(p):
            pltpu.make_async_copy(bufs[p], acc.at[ldst.at[p]],
                                  csems[p]).wait()

        def scale(p):
            buf = bufs[p]
            for gq in range(chunk // L):
                vals16 = vring[p, pl.ds(gq * L, L)]
                for i in range(L):
                    v = vals16[i]
                    row = gq * L + i
                    for g in range(d // L):
                        sl = pl.ds(g * L, L)
                        buf[row, sl] = buf[row, sl] * v

        def quad_body(jj, carry):
            for p in range(4):
                j = jj * 4 + p
                wait_gather(p)
                for q in range(chunk // L):
                    sl = pl.ds(q * L, L)
                    ldst[p, sl] = dring[p, sl]
                p2 = (p + 2) % 4

                @pl.when(j + 2 < nch)
                def _():
                    wait_slab(j + 2, p2)

                    @pl.when(j >= 2)
                    def _():
                        wait_scatter(p2)
                    start_gather(p2)
                scale(p)
                start_scatter(p)
                @pl.when(j + 4 < nch)
                def _():
                    start_slab(j + 4, p)
            return carry

        for p in range(4):
            start_slab(p, p)
        wait_slab(0, 0)
        start_gather(0)
        wait_slab(1, 1)
        start_gather(1)
        lax.fori_loop(0, nch // 4, quad_body, 0)
        for p in range(4):
            wait_scatter(p)

        plsc.subcore_barrier()
        pltpu.sync_copy(acc.at[pl.ds(r0, rows_per_tile)],
                        part_hbm.at[c, pl.ds(r0, rows_per_tile)])

    return sc_kernel


def _tc_body(p_ref, w_ref, b_ref, g_ref, be_ref, o_ref):
    z = lax.dot_general(p_ref[0] + p_ref[1], w_ref[...],
                        (((1,), (1,)), ((), ())),
                        preferred_element_type=jnp.float32)
    z = z + b_ref[...]
    mu = jnp.mean(z, axis=-1, keepdims=True)
    zc = z - mu
    var = jnp.mean(zc * zc, axis=-1, keepdims=True)
    zn = zc * lax.rsqrt(var + 1e-5) * g_ref[...] + be_ref[...]
    o_ref[...] = zn * 0.5 * (1.0 + lax.erf(zn * (1.0 / math.sqrt(2.0))))


def kernel(x, edge_index, edge_values, W, b, gamma, beta):
    B, n, d_in = x.shape
    d = B * d_in
    d_out = W.shape[0]
    e = edge_values.shape[0]
    x2 = jnp.transpose(x.astype(jnp.float32), (1, 0, 2)).reshape(n, d)

    chunk = 32
    nch = -(-(-(-e // (NW * chunk))) // 4) * 4
    e_pad = NW * nch * chunk
    src = jnp.concatenate(
        [edge_index[1], jnp.zeros((e_pad - e,), jnp.int32)]).reshape(
            NW, nch, chunk)
    dst = jnp.concatenate(
        [edge_index[0], jnp.zeros((e_pad - e,), jnp.int32)]).reshape(
            NW, nch, chunk)
    val = jnp.concatenate(
        [edge_values, jnp.zeros((e_pad - e,), jnp.float32)]).reshape(
            NW, nch, chunk)

    r_pad = ((n + NS * 8 - 1) // (NS * 8)) * (NS * 8)
    zeros = jnp.zeros((r_pad, d), jnp.float32)

    sc = _make_sc_aggregate(r_pad, d, nch, chunk)
    partials = sc(x2, src, dst, val, zeros)

    blk = 2000
    out = pl.pallas_call(
        _tc_body,
        grid=(n // blk,),
        in_specs=[
            pl.BlockSpec((NC, blk, d), lambda i: (0, i, 0)),
            pl.BlockSpec((d_out, d), lambda i: (0, 0)),
            pl.BlockSpec((1, d_out), lambda i: (0, 0)),
            pl.BlockSpec((1, d_out), lambda i: (0, 0)),
            pl.BlockSpec((1, d_out), lambda i: (0, 0)),
        ],
        out_specs=pl.BlockSpec((blk, d_out), lambda i: (i, 0)),
        out_shape=jax.ShapeDtypeStruct((n, d_out), jnp.float32),
    )(partials, W, b.reshape(1, d_out), gamma.reshape(1, d_out),
      beta.reshape(1, d_out))

    return out.reshape(n, B, d_out).transpose(1, 0, 2)
